# Initial kernel scaffold; baseline (speedup 1.0000x reference)
import jax
import jax.numpy as jnp
from jax.experimental import pallas as pl


def kernel(node_x, edge_x, sources, targets, features, W, b):
    raise NotImplementedError("write your pallas kernel here")


# speedup vs baseline: 7.2880x; 7.2880x over previous
import functools

import jax
import jax.numpy as jnp
from jax import lax
from jax.experimental import pallas as pl
from jax.experimental.pallas import tpu as pltpu
from jax.experimental.pallas import tpu_sc as plsc

NUM_NODES = 10000
NUM_EDGES = 320000
D = 128

NC, NS = 2, 16
NW = NC * NS
E_W = NUM_EDGES // NW
K = 80
NSTAGE = 5
CPS = 25
ACC_ROWS = 10240
ROWS_PER_TILE = ACC_ROWS // NS


def _sc_aggregate(node_x, src4, tgt4):
    mesh = plsc.VectorSubcoreMesh(core_axis_name="c", subcore_axis_name="s")

    @functools.partial(
        pl.kernel,
        out_type=jax.ShapeDtypeStruct((NC * ACC_ROWS, D), jnp.float32),
        mesh=mesh,
        scratch_types=[
            pltpu.VMEM((CPS, K), jnp.int32),
            pltpu.VMEM((CPS, K), jnp.int32),
            pltpu.VMEM((K, D), jnp.float32),
            pltpu.VMEM_SHARED((ACC_ROWS, D), jnp.float32),
            pltpu.SemaphoreType.DMA,
        ],
    )
    def agg_kernel(node_hbm, src_hbm, tgt_hbm, out_hbm,
                   src_v, tgt_v, rows_v, acc_sh, sem):
        cid = lax.axis_index("c")
        sid = lax.axis_index("s")
        wid = sid * NC + cid

        def zrow(r, carry):
            for c16 in range(D // 16):
                rows_v[r, pl.ds(c16 * 16, 16)] = jnp.zeros((16,), jnp.float32)
            return carry
        lax.fori_loop(0, K, zrow, 0)
        for t in range(ROWS_PER_TILE // K):
            pltpu.sync_copy(
                rows_v, acc_sh.at[pl.ds(sid * ROWS_PER_TILE + t * K, K)])
        plsc.subcore_barrier()

        def stage(s, carry):
            pltpu.sync_copy(src_hbm.at[wid, s], src_v)
            pltpu.sync_copy(tgt_hbm.at[wid, s], tgt_v)

            def chunk(j, carry2):
                pltpu.async_copy(node_hbm.at[src_v.at[j]], rows_v, sem).wait()
                pltpu.sync_copy(rows_v, acc_sh.at[tgt_v.at[j]], add=True)
                return carry2
            return lax.fori_loop(0, CPS, chunk, carry)
        lax.fori_loop(0, NSTAGE, stage, 0)
        plsc.subcore_barrier()

        base = cid * ACC_ROWS + sid * ROWS_PER_TILE
        pltpu.sync_copy(
            acc_sh.at[pl.ds(sid * ROWS_PER_TILE, ROWS_PER_TILE)],
            out_hbm.at[pl.ds(base, ROWS_PER_TILE)])

    return agg_kernel(node_x, src4, tgt4)


def _dense(partials, node_x, W, b2):
    BR = 1000

    def body(p_ref, x_ref, w_ref, b_ref, o_ref):
        agg = p_ref[0] + p_ref[1]
        acc = jnp.dot(agg, w_ref[:D, :], preferred_element_type=jnp.float32,
                      precision=lax.Precision.HIGHEST)
        acc += jnp.dot(x_ref[...], w_ref[D:, :],
                       preferred_element_type=jnp.float32,
                       precision=lax.Precision.HIGHEST)
        o_ref[...] = acc + b_ref[...]

    return pl.pallas_call(
        body,
        grid=(NUM_NODES // BR,),
        in_specs=[
            pl.BlockSpec((2, BR, D), lambda i: (0, i, 0)),
            pl.BlockSpec((BR, D), lambda i: (i, 0)),
            pl.BlockSpec((2 * D, D), lambda i: (0, 0)),
            pl.BlockSpec((1, D), lambda i: (0, 0)),
        ],
        out_specs=pl.BlockSpec((BR, D), lambda i: (i, 0)),
        out_shape=jax.ShapeDtypeStruct((NUM_NODES, D), jnp.float32),
    )(partials, node_x, W, b2)


def kernel(node_x, edge_x, sources, targets, features, W, b):
    del edge_x, features
    src4 = sources.astype(jnp.int32).reshape(NW, NSTAGE, CPS, K)
    tgt4 = targets.astype(jnp.int32).reshape(NW, NSTAGE, CPS, K)
    partials = _sc_aggregate(node_x, src4, tgt4)
    partials = partials.reshape(NC, ACC_ROWS, D)[:, :NUM_NODES, :]
    return _dense(partials, node_x, W, b.reshape(1, D))
